# baseline (speedup 1.0000x reference)
import functools

import jax
import jax.numpy as jnp
from jax import lax
from jax.experimental import pallas as pl
from jax.experimental.pallas import tpu as pltpu
from jax.experimental.pallas import tpu_sc as plsc

BATCH = 16384
EDIM = 64
RDIM = 32
NREL = 1000
NC = 2
NS = 16
NW = NC * NS
ROWS_PER_TILE = BATCH // NW
CHUNK = 32
N_CHUNKS = ROWS_PER_TILE // CHUNK
_PIB = jax.lax.GatherScatterMode.PROMISE_IN_BOUNDS
_GATHER_DNUMS = lax.GatherDimensionNumbers(
    offset_dims=(), collapsed_slice_dims=(0,), start_index_map=(0,))


def _bcast(vec, lane):
    idx = jnp.full((16, 1), lane, dtype=jnp.int32)
    return lax.gather(vec, idx, _GATHER_DNUMS, slice_sizes=(1,), mode=_PIB)


def _sc_body(h_hbm, r_hbm, p_hbm, n_hbm, ent_hbm, rel_hbm, w_hbm,
             diff_hbm, reg_hbm,
             h_idx, r_idx, p_idx, n_idx,
             h_buf, p_buf, n_buf, re_buf, w_buf, score_buf, reg_stage, sem):
    wid = lax.axis_index("s") * NC + lax.axis_index("c")
    base = wid * ROWS_PER_TILE
    zeros = jnp.zeros((16,), jnp.float32)

    def chunk_body(c, carry):
        racc0, racc1 = carry
        row0 = base + c * CHUNK
        pltpu.sync_copy(h_hbm.at[pl.ds(row0, CHUNK)], h_idx)
        pltpu.sync_copy(r_hbm.at[pl.ds(row0, CHUNK)], r_idx)
        pltpu.sync_copy(p_hbm.at[pl.ds(row0, CHUNK)], p_idx)
        pltpu.sync_copy(n_hbm.at[pl.ds(row0, CHUNK)], n_idx)
        c1 = pltpu.async_copy(ent_hbm.at[h_idx], h_buf, sem)
        c2 = pltpu.async_copy(ent_hbm.at[p_idx], p_buf, sem)
        c3 = pltpu.async_copy(ent_hbm.at[n_idx], n_buf, sem)
        c4 = pltpu.async_copy(rel_hbm.at[r_idx], re_buf, sem)
        c5 = pltpu.async_copy(w_hbm.at[r_idx], w_buf, sem)
        c1.wait(); c2.wait(); c3.wait(); c4.wait(); c5.wait()

        def row_body(i, rc):
            racc0, racc1 = rc
            uq = []
            vq = []
            for q in range(4):
                hq = h_buf[i, pl.ds(q * 16, 16)]
                pq = p_buf[i, pl.ds(q * 16, 16)]
                nq = n_buf[i, pl.ds(q * 16, 16)]
                uq.append(hq - pq)
                vq.append(hq - nq)
                racc0 = racc0 + hq * hq + pq * pq + nq * nq
            re0 = re_buf[i, pl.ds(0, 16)]
            re1 = re_buf[i, pl.ds(16, 16)]
            racc1 = racc1 + re0 * re0 + re1 * re1
            ap0 = re0
            ap1 = re1
            an0 = re0
            an1 = re1
            for d in range(EDIM):
                w0 = w_buf[i, pl.ds(d * RDIM, 16)]
                w1 = w_buf[i, pl.ds(d * RDIM + 16, 16)]
                ub = _bcast(uq[d // 16], d % 16)
                vb = _bcast(vq[d // 16], d % 16)
                ap0 = ap0 + ub * w0
                ap1 = ap1 + ub * w1
                an0 = an0 + vb * w0
                an1 = an1 + vb * w1
            spn = ap0 * ap0 + ap1 * ap1 - an0 * an0 - an1 * an1
            score_buf[i, :] = spn
            return (racc0, racc1)

        racc0, racc1 = lax.fori_loop(0, CHUNK, row_body, (racc0, racc1))
        pltpu.sync_copy(score_buf, diff_hbm.at[pl.ds(row0, CHUNK)])
        return (racc0, racc1)

    racc0, racc1 = lax.fori_loop(0, N_CHUNKS, chunk_body, (zeros, zeros))
    reg_stage[pl.ds(0, 16)] = racc0
    reg_stage[pl.ds(16, 16)] = racc1
    pltpu.sync_copy(reg_stage, reg_hbm.at[wid])


def _tc_body(diff_ref, reg_ref, out_ref):
    spn = diff_ref[...]
    z = 0.5 * jnp.sum(spn, axis=1, keepdims=True)
    nz = -z
    softplus = jnp.maximum(nz, 0.0) + jnp.log1p(jnp.exp(-jnp.abs(nz)))
    kg = jnp.sum(softplus) * (1.0 / BATCH)
    regt = jnp.sum(reg_ref[...]) * (1.0 / (2.0 * BATCH))
    out_ref[0, 0] = kg + 0.01 * regt


def kernel(h, r, pos_t, neg_t, entity_embed, relation_embed, relation_weight):
    w2 = relation_weight.reshape(NREL, EDIM * RDIM)
    mesh = plsc.VectorSubcoreMesh(core_axis_name="c", subcore_axis_name="s")
    sc = pl.kernel(
        _sc_body,
        mesh=mesh,
        compiler_params=pltpu.CompilerParams(use_tc_tiling_on_sc=False),
        out_type=(
            jax.ShapeDtypeStruct((BATCH, 16), jnp.float32),
            jax.ShapeDtypeStruct((NW, 32), jnp.float32),
        ),
        scratch_types=[
            pltpu.VMEM((CHUNK,), jnp.int32),
            pltpu.VMEM((CHUNK,), jnp.int32),
            pltpu.VMEM((CHUNK,), jnp.int32),
            pltpu.VMEM((CHUNK,), jnp.int32),
            pltpu.VMEM((CHUNK, EDIM), jnp.float32),
            pltpu.VMEM((CHUNK, EDIM), jnp.float32),
            pltpu.VMEM((CHUNK, EDIM), jnp.float32),
            pltpu.VMEM((CHUNK, RDIM), jnp.float32),
            pltpu.VMEM((CHUNK, EDIM * RDIM), jnp.float32),
            pltpu.VMEM((CHUNK, 16), jnp.float32),
            pltpu.VMEM((32,), jnp.float32),
            pltpu.SemaphoreType.DMA,
        ],
    )
    diff, reg = sc(h, r, pos_t, neg_t, entity_embed, relation_embed, w2)
    out = pl.pallas_call(
        _tc_body,
        out_shape=jax.ShapeDtypeStruct((1, 1), jnp.float32),
        out_specs=pl.BlockSpec(memory_space=pltpu.SMEM),
    )(diff, reg)
    return out[0, 0]
